# Initial kernel scaffold; baseline (speedup 1.0000x reference)
import jax
import jax.numpy as jnp
from jax.experimental import pallas as pl


def kernel(coordinates, mask, W_edge, b_edge):
    raise NotImplementedError("write your pallas kernel here")


# speedup vs baseline: 22.8773x; 22.8773x over previous
import math

import jax
import jax.numpy as jnp
from jax.experimental import pallas as pl

_L = 1024
_K = 32
_NB = 37
_R = 64


def _norm3(v):
    n = jnp.sqrt(jnp.sum(v * v, axis=1, keepdims=True))
    return v / jnp.clip(n, 1e-12, None)


def _cross3(u, v):
    u0, u1, u2 = u[:, 0:1], u[:, 1:2], u[:, 2:3]
    v0, v1, v2 = v[:, 0:1], v[:, 1:2], v[:, 2:3]
    return jnp.concatenate(
        [u1 * v2 - u2 * v1, u2 * v0 - u0 * v2, u0 * v1 - u1 * v0], axis=1
    )


def _vcb_rows(cl):
    n = cl[:, 0:3]
    ca = cl[:, 3:6]
    c = cl[:, 6:9]
    ca_n = _norm3(n - ca)
    ca_c = _norm3(c - ca)
    bis = _norm3(ca_n + ca_c)
    perp = _norm3(_cross3(ca_n, ca_c))
    cb_dir = _norm3(-bis + 0.5 * perp)
    return ca + 1.54 * cb_dir


def _vcb_lanes(ct):
    def norm1(x, y, z):
        n = jnp.sqrt(x * x + y * y + z * z)
        n = jnp.clip(n, 1e-12, None)
        return x / n, y / n, z / n

    nx, ny, nz = ct[0:1], ct[1:2], ct[2:3]
    cax, cay, caz = ct[3:4], ct[4:5], ct[5:6]
    cx, cy, cz = ct[6:7], ct[7:8], ct[8:9]
    anx, any_, anz = norm1(nx - cax, ny - cay, nz - caz)
    acx, acy, acz = norm1(cx - cax, cy - cay, cz - caz)
    bx, by, bz = norm1(anx + acx, any_ + acy, anz + acz)
    crx = any_ * acz - anz * acy
    cry = anz * acx - anx * acz
    crz = anx * acy - any_ * acx
    px, py, pz = norm1(crx, cry, crz)
    dx, dy, dz = norm1(-bx + 0.5 * px, -by + 0.5 * py, -bz + 0.5 * pz)
    return cax + 1.54 * dx, cay + 1.54 * dy, caz + 1.54 * dz


def _body(cl_ref, clr_ref, ct_ref, mcol_ref, mrow_ref, w185_ref, wsin_ref,
          wcos_ref, b_ref, lo_ref, hi_ref, div_ref, ef_ref, nbr_ref):
    r0 = pl.program_id(1) * _R
    cl = cl_ref[0]
    clr = clr_ref[0]
    ct = ct_ref[0]
    mcol = mcol_ref[0]
    maskr = mrow_ref[0]

    vcb = _vcb_rows(cl)
    vxc, vyc, vzc = _vcb_lanes(ct)

    cols = []
    for d in range(3):
        for a in range(4):
            cols.append(cl[:, 3 * a + d:3 * a + d + 1])
        cols.append(vcb[:, d:d + 1])
    C = jnp.concatenate(cols, axis=1)

    vcbr = _vcb_rows(clr)
    rcols = []
    for d in range(3):
        for a in range(4):
            rcols.append(clr[:, 3 * a + d:3 * a + d + 1])
        rcols.append(vcbr[:, d:d + 1])
    rowC = jnp.concatenate(rcols, axis=1)
    vxr = rowC[:, 4:5]
    vyr = rowC[:, 9:10]
    vzr = rowC[:, 14:15]

    d2 = (vxr - vxc) ** 2 + (vyr - vyc) ** 2 + (vzr - vzc) ** 2
    inf = jnp.float32(jnp.inf)
    d2m = jnp.where(mcol > 0.5, jnp.sqrt(d2), inf)
    iotaf = jax.lax.broadcasted_iota(jnp.int32, (1, _L), 1).astype(jnp.float32)
    picks = []
    for t in range(_K + 1):
        m = jnp.min(d2m, axis=1, keepdims=True)
        cand = jnp.where(d2m <= m, jnp.broadcast_to(iotaf, d2m.shape),
                         jnp.float32(1e9))
        amin = jnp.min(cand, axis=1, keepdims=True)
        d2m = jnp.where(iotaf == amin, inf, d2m)
        if t > 0:
            picks.append(amin)
    nbr_f = jnp.concatenate(picks, axis=1)

    iota_l3 = jax.lax.broadcasted_iota(jnp.int32, (_R, _K, _L), 2).astype(jnp.float32)
    N3 = (nbr_f[:, :, None] == iota_l3).astype(jnp.float32)
    N2 = N3.reshape(_R * _K, _L)
    G = jnp.dot(N2, C, preferred_element_type=jnp.float32,
                precision=jax.lax.Precision.HIGHEST)
    rowE = jnp.broadcast_to(rowC[:, None, :], (_R, _K, 15)).reshape(
        _R * _K, 15)
    diff = rowE - G
    dsq = diff * diff
    nd2 = dsq[:, 0:5] + dsq[:, 5:10] + dsq[:, 10:15]
    nd = jnp.sqrt(nd2)

    lo = lo_ref[...]
    hi = hi_ref[...]
    ohs = []
    for a in range(5):
        na = nd[:, a:a + 1]
        ohs.append(((na > lo) & (na <= hi)).astype(jnp.float32))
    rowids = jnp.float32(r0) + jax.lax.broadcasted_iota(
        jnp.int32, (_R, 1), 0).astype(jnp.float32)
    rel = jnp.clip(nbr_f - rowids, -32.0, 32.0) + 32.0
    iota65 = jax.lax.broadcasted_iota(jnp.int32, (_R, _K, 65), 2).astype(jnp.float32)
    ohrel = (rel[:, :, None] == iota65).astype(jnp.float32).reshape(
        _R * _K, 65)
    OH = jnp.concatenate(ohs + [ohrel], axis=1)

    rv = jax.lax.broadcasted_iota(jnp.int32, (65, 1), 0).astype(jnp.float32) - 32.0
    ang = rv * div_ref[...]
    Tpe = (jnp.dot(jnp.sin(ang), wsin_ref[...],
                   preferred_element_type=jnp.float32,
                   precision=jax.lax.Precision.HIGHEST)
           + jnp.dot(jnp.cos(ang), wcos_ref[...],
                     preferred_element_type=jnp.float32,
                     precision=jax.lax.Precision.HIGHEST)
           + b_ref[...])
    table = jnp.concatenate([w185_ref[...], Tpe], axis=0)

    ef2 = jnp.dot(OH, table, preferred_element_type=jnp.float32,
                  precision=jax.lax.Precision.HIGHEST)
    mrk = jnp.broadcast_to(maskr[:, None, :], (_R, _K, 1)).reshape(
        _R * _K, 1)
    ef2 = ef2 * mrk

    ef_ref[...] = ef2.reshape(1, _R, _K, 64)
    nbr_ref[...] = nbr_f.astype(jnp.int32).reshape(1, _R, _K)


def kernel(coordinates, mask, W_edge, b_edge):
    b, l = coordinates.shape[:2]
    cl = coordinates.reshape(b, l, 12)
    ct = jnp.transpose(cl, (0, 2, 1))
    maskf = mask.astype(jnp.float32)[:, None, :]
    mrow = mask.astype(jnp.float32)[:, :, None]
    w185 = W_edge[:185]
    wpe = W_edge[185:]
    wsin = wpe[0::2]
    wcos = wpe[1::2]
    bins = jnp.linspace(2.0, 20.0, _NB)
    hi = jnp.concatenate([bins[:_NB - 1], jnp.array([jnp.inf])])[None, :]
    lo = jnp.concatenate([jnp.array([-jnp.inf]), bins[:_NB - 1]])[None, :]
    div = jnp.exp(jnp.arange(0, 64, 2).astype(jnp.float32)
                  * (-(math.log(10000.0) / 64)))[None, :]
    b2 = b_edge[None, :]

    grid = (b, l // _R)
    ef, nbr = pl.pallas_call(
        _body,
        grid=grid,
        in_specs=[
            pl.BlockSpec((1, l, 12), lambda i, j: (i, 0, 0)),
            pl.BlockSpec((1, _R, 12), lambda i, j: (i, j, 0)),
            pl.BlockSpec((1, 12, l), lambda i, j: (i, 0, 0)),
            pl.BlockSpec((1, 1, l), lambda i, j: (i, 0, 0)),
            pl.BlockSpec((1, _R, 1), lambda i, j: (i, j, 0)),
            pl.BlockSpec((185, 64), lambda i, j: (0, 0)),
            pl.BlockSpec((32, 64), lambda i, j: (0, 0)),
            pl.BlockSpec((32, 64), lambda i, j: (0, 0)),
            pl.BlockSpec((1, 64), lambda i, j: (0, 0)),
            pl.BlockSpec((1, _NB), lambda i, j: (0, 0)),
            pl.BlockSpec((1, _NB), lambda i, j: (0, 0)),
            pl.BlockSpec((1, 32), lambda i, j: (0, 0)),
        ],
        out_specs=[
            pl.BlockSpec((1, _R, _K, 64), lambda i, j: (i, j, 0, 0)),
            pl.BlockSpec((1, _R, _K), lambda i, j: (i, j, 0)),
        ],
        out_shape=[
            jax.ShapeDtypeStruct((b, l, _K, 64), jnp.float32),
            jax.ShapeDtypeStruct((b, l, _K), jnp.int32),
        ],
    )(cl, cl, ct, maskf, mrow, w185, wsin, wcos, b2, lo, hi, div)
    return ef, nbr
